# baseline (speedup 1.0000x reference)
import functools

import jax
import jax.numpy as jnp
from jax import lax
from jax.experimental import pallas as pl
from jax.experimental.pallas import tpu as pltpu
from jax.experimental.pallas import tpu_sc as plsc

B, S, D = 4, 4096, 1024
NUM_EMB = 4

ROWS = B * S
NW = 32
ROWS_PER_TILE = ROWS // NW
BUF_ROWS = 64
N_WRITES = ROWS_PER_TILE // BUF_ROWS
LANES = 16
CHUNKS = D // LANES


def _sc_body(idx_hbm, table_hbm, out_hbm, idx_v, buf, gsem, wsem):
    wid = lax.axis_index("s") * 2 + lax.axis_index("c")
    base = wid * ROWS_PER_TILE
    pltpu.sync_copy(idx_hbm, idx_v)
    pltpu.async_copy(table_hbm.at[idx_v], buf.at[pl.ds(0, 8)], gsem).wait()

    for c in range(CHUNKS):
        col = c * LANES
        v = buf[0, pl.ds(col, LANES)]

        def _fill(g, _, col=col, v=v):
            base_row = 8 * g
            for k in range(8):
                buf[base_row + k, pl.ds(col, LANES)] = v
            return _

        lax.fori_loop(1, BUF_ROWS // 8, _fill, 0)

    copies = [
        pltpu.async_copy(buf, out_hbm.at[pl.ds(base + j * BUF_ROWS, BUF_ROWS)], wsem)
        for j in range(N_WRITES)
    ]
    for c in copies:
        c.wait()


@functools.partial(
    pl.kernel,
    out_type=jax.ShapeDtypeStruct((ROWS, D), jnp.float32),
    mesh=plsc.VectorSubcoreMesh(core_axis_name="c", subcore_axis_name="s"),
    scratch_types=[
        pltpu.VMEM((8,), jnp.int32),
        pltpu.VMEM((BUF_ROWS, D), jnp.float32),
        pltpu.SemaphoreType.DMA,
        pltpu.SemaphoreType.DMA,
    ],
)
def _sc_broadcast(idx_hbm, table_hbm, out_hbm, idx_v, buf, gsem, wsem):
    _sc_body(idx_hbm, table_hbm, out_hbm, idx_v, buf, gsem, wsem)


def kernel(x, modality_index, emb_table):
    del x
    idx_vec = jnp.full((8,), modality_index, dtype=jnp.int32)
    out = _sc_broadcast(idx_vec, emb_table)
    return out.reshape(B, S, D)


# speedup vs baseline: 1.5239x; 1.5239x over previous
import functools

import jax
import jax.numpy as jnp
from jax import lax
from jax.experimental import pallas as pl
from jax.experimental.pallas import tpu as pltpu
from jax.experimental.pallas import tpu_sc as plsc

B, S, D = 4, 4096, 1024
NUM_EMB = 4

ROWS = B * S
NW = 32
ROWS_PER_TILE = ROWS // NW
BUF_ROWS = 64
N_WRITES = ROWS_PER_TILE // BUF_ROWS


def _sc_body(idx_hbm, table_hbm, out_hbm, idx_v, buf, shared, gsem, wsem):
    sid = lax.axis_index("s")
    wid = sid * 2 + lax.axis_index("c")
    base = wid * ROWS_PER_TILE

    @pl.when(sid == 0)
    def _stage():
        pltpu.sync_copy(idx_hbm, idx_v)
        pltpu.async_copy(table_hbm.at[idx_v], buf, gsem).wait()
        for k in range(BUF_ROWS // 8):
            pltpu.sync_copy(buf, shared.at[pl.ds(8 * k, 8)])

    plsc.subcore_barrier()
    copies = [
        pltpu.async_copy(
            shared, out_hbm.at[pl.ds(base + j * BUF_ROWS, BUF_ROWS)], wsem
        )
        for j in range(N_WRITES)
    ]
    for c in copies:
        c.wait()


@functools.partial(
    pl.kernel,
    out_type=jax.ShapeDtypeStruct((ROWS, D), jnp.float32),
    mesh=plsc.VectorSubcoreMesh(core_axis_name="c", subcore_axis_name="s"),
    scratch_types=[
        pltpu.VMEM((8,), jnp.int32),
        pltpu.VMEM((8, D), jnp.float32),
        pltpu.VMEM_SHARED((BUF_ROWS, D), jnp.float32),
        pltpu.SemaphoreType.DMA,
        pltpu.SemaphoreType.DMA,
    ],
)
def _sc_broadcast(idx_hbm, table_hbm, out_hbm, idx_v, buf, shared, gsem, wsem):
    _sc_body(idx_hbm, table_hbm, out_hbm, idx_v, buf, shared, gsem, wsem)


def kernel(x, modality_index, emb_table):
    del x
    idx_vec = jnp.full((8,), modality_index, dtype=jnp.int32)
    out = _sc_broadcast(idx_vec, emb_table)
    return out.reshape(B, S, D)
